# baseline (speedup 1.0000x reference)
import functools

import jax
import jax.numpy as jnp
from jax import lax
from jax.experimental import pallas as pl
from jax.experimental.pallas import tpu as pltpu
from jax.experimental.pallas import tpu_sc as plsc

EMBED_DIM = 512
NIMG = 256
NPOS = 1025
NC = 2
NS = 16
NW = NC * NS
IPW = NIMG // NW
CHUNK = 64
CPI = 1024 // CHUNK
NCHUNK = IPW * CPI
NBUF = 3

_mesh = plsc.VectorSubcoreMesh(
    core_axis_name="c", subcore_axis_name="s", num_cores=NC, num_subcores=NS
)


@functools.partial(
    pl.kernel,
    out_type=jax.ShapeDtypeStruct((NIMG, NPOS, EMBED_DIM), jnp.float32),
    mesh=_mesh,
    scratch_types=[
        pltpu.VMEM((NCHUNK, CHUNK), jnp.int32),
        pltpu.VMEM((NBUF, CHUNK, EMBED_DIM), jnp.float32),
        pltpu.VMEM((IPW * 8,), jnp.int32),
        pltpu.SemaphoreType.DMA,
        pltpu.SemaphoreType.DMA,
        pltpu.SemaphoreType.DMA,
        pltpu.SemaphoreType.DMA,
        pltpu.SemaphoreType.DMA,
        pltpu.SemaphoreType.DMA,
        pltpu.SemaphoreType.DMA,
    ],
)
def _sc_gather(idx_hbm, tidx_hbm, table_hbm, out_hbm,
               idx_v, rows_v, tidx_v,
               g0, g1, g2, o0, o1, o2, tsem):
    wid = lax.axis_index("s") * NC + lax.axis_index("c")
    img0 = wid * IPW
    gsem = (g0, g1, g2)
    osem = (o0, o1, o2)

    pltpu.sync_copy(idx_hbm.at[wid], idx_v)
    pltpu.sync_copy(tidx_hbm.at[wid], tidx_v)

    def gather_start(k, b):
        pltpu.make_async_copy(
            table_hbm.at[idx_v.at[k]], rows_v.at[b], gsem[b]
        ).start()

    def gather_wait(b):
        pltpu.make_async_copy(
            table_hbm.at[pl.ds(0, CHUNK)], rows_v.at[b], gsem[b]
        ).wait()

    def out_start(k, b):
        img = img0 + k // CPI
        r0 = (k % CPI) * CHUNK
        pltpu.make_async_copy(
            rows_v.at[b], out_hbm.at[img, pl.ds(r0, CHUNK)], osem[b]
        ).start()

    def out_wait(b):
        pltpu.make_async_copy(
            table_hbm.at[pl.ds(0, CHUNK)], rows_v.at[b], osem[b]
        ).wait()

    for b in range(NBUF):
        gather_start(b, b)

    def body(kk, carry):
        k0 = kk * NBUF
        for b in range(NBUF):
            gather_wait(b)
            out_start(k0 + b, b)
        for b in range(NBUF):

            @pl.when(k0 + b + NBUF < NCHUNK)
            def _():
                out_wait(b)
                gather_start(k0 + b + NBUF, b)

        return carry

    lax.fori_loop(0, NCHUNK // NBUF, body, 0)
    rem = NCHUNK - (NCHUNK // NBUF) * NBUF
    for b in range(rem):
        gather_wait(b)
        out_start(NCHUNK - rem + b, b)
    for b in range(NBUF):
        out_wait(b)

    pltpu.make_async_copy(table_hbm.at[tidx_v], rows_v.at[0], tsem).start()
    pltpu.make_async_copy(
        table_hbm.at[pl.ds(0, IPW * 8)], rows_v.at[0], tsem
    ).wait()
    tail_off = pl.multiple_of((wid * 0 + 128) * 8, 8)
    for i in range(IPW):
        pltpu.make_async_copy(
            rows_v.at[0, pl.ds(8 * i, 8)],
            out_hbm.at[img0 + i, pl.ds(tail_off, 8)],
            osem[i % NBUF],
        ).start()
    for i in range(IPW):
        pltpu.make_async_copy(
            table_hbm.at[pl.ds(0, 8)],
            rows_v.at[0, pl.ds(0, 8)],
            osem[i % NBUF],
        ).wait()


def kernel(idx, table):
    idx_i32 = idx.astype(jnp.int32)
    idx_main = idx_i32[:, :1024].reshape(NW, NCHUNK, CHUNK)
    idx_tail = jnp.zeros((NW, IPW, 8), jnp.int32)
    idx_tail = idx_tail.at[:, :, 0].set(idx_i32[:, 1024].reshape(NW, IPW))
    return _sc_gather(idx_main, idx_tail.reshape(NW, IPW * 8), table)


# speedup vs baseline: 1.0862x; 1.0862x over previous
import functools

import jax
import jax.numpy as jnp
from jax import lax
from jax.experimental import pallas as pl
from jax.experimental.pallas import tpu as pltpu
from jax.experimental.pallas import tpu_sc as plsc

EMBED_DIM = 512
NIMG = 256
NPOS = 1025
NC = 2
NS = 16
NW = NC * NS
PPW = 33
CHUNK = 64
CPP = NIMG // CHUNK
NCHUNK = PPW * CPP
NBUF = 3

_mesh = plsc.VectorSubcoreMesh(
    core_axis_name="c", subcore_axis_name="s", num_cores=NC, num_subcores=NS
)


@functools.partial(
    pl.kernel,
    out_type=jax.ShapeDtypeStruct((NPOS, NIMG, EMBED_DIM), jnp.float32),
    mesh=_mesh,
    scratch_types=[
        pltpu.VMEM((PPW, CPP, CHUNK), jnp.int32),
        pltpu.VMEM((NBUF, CHUNK, EMBED_DIM), jnp.float32),
        pltpu.SemaphoreType.DMA,
        pltpu.SemaphoreType.DMA,
        pltpu.SemaphoreType.DMA,
        pltpu.SemaphoreType.DMA,
        pltpu.SemaphoreType.DMA,
        pltpu.SemaphoreType.DMA,
    ],
)
def _sc_gather(idx_hbm, table_hbm, out_hbm,
               idx_v, rows_v,
               g0, g1, g2, o0, o1, o2):
    wid = lax.axis_index("s") * NC + lax.axis_index("c")
    p0 = wid * PPW
    gsem = (g0, g1, g2)
    osem = (o0, o1, o2)

    pltpu.sync_copy(idx_hbm.at[wid], idx_v)

    def valid(k):
        return p0 + k // CPP < NPOS

    def gather_start(k, b):
        pltpu.make_async_copy(
            table_hbm.at[idx_v.at[k // CPP, k % CPP]], rows_v.at[b], gsem[b]
        ).start()

    def gather_wait(b):
        pltpu.make_async_copy(
            table_hbm.at[pl.ds(0, CHUNK)], rows_v.at[b], gsem[b]
        ).wait()

    def out_start(k, b):
        p = p0 + k // CPP
        b0 = (k % CPP) * CHUNK
        pltpu.make_async_copy(
            rows_v.at[b], out_hbm.at[p, pl.ds(b0, CHUNK)], osem[b]
        ).start()

    def out_wait(b):
        pltpu.make_async_copy(
            table_hbm.at[pl.ds(0, CHUNK)], rows_v.at[b], osem[b]
        ).wait()

    for b in range(NBUF):
        gather_start(b, b)

    def body(kk, carry):
        k0 = kk * NBUF
        for b in range(NBUF):
            gather_wait(b)

            @pl.when(valid(k0 + b))
            def _():
                out_start(k0 + b, b)

        for b in range(NBUF):

            @pl.when(k0 + b + NBUF < NCHUNK)
            def _():
                @pl.when(valid(k0 + b))
                def _():
                    out_wait(b)

                gather_start(k0 + b + NBUF, b)

        return carry

    lax.fori_loop(0, NCHUNK // NBUF, body, 0)
    for b in range(NBUF):
        k = NCHUNK - NBUF + b

        @pl.when(valid(k))
        def _():
            out_wait(b)


def kernel(idx, table):
    idx_t = idx.astype(jnp.int32).T
    idx_t = jnp.pad(idx_t, ((0, NW * PPW - NPOS), (0, 0)))
    idx_r = idx_t.reshape(NW, PPW, CPP, CHUNK)
    out_t = _sc_gather(idx_r, table)
    return jnp.transpose(out_t, (1, 0, 2))


# speedup vs baseline: 2.1943x; 2.0202x over previous
import functools

import jax
import jax.numpy as jnp
from jax import lax
from jax.experimental import pallas as pl
from jax.experimental.pallas import tpu as pltpu
from jax.experimental.pallas import tpu_sc as plsc

EMBED_DIM = 512
NIMG = 256
NPOS = 1025
NC = 2
NS = 16
NW = NC * NS
PPW = 33
CHUNK = 64
CPP = NIMG // CHUNK
NCHUNK = PPW * CPP
NBUF = 3

_mesh = plsc.VectorSubcoreMesh(
    core_axis_name="c", subcore_axis_name="s", num_cores=NC, num_subcores=NS
)


@functools.partial(
    pl.kernel,
    out_type=jax.ShapeDtypeStruct((NPOS, NIMG, EMBED_DIM), jnp.float32),
    mesh=_mesh,
    scratch_types=[
        pltpu.VMEM((PPW, CPP, CHUNK), jnp.int32),
        pltpu.VMEM((NBUF, CHUNK, EMBED_DIM), jnp.float32),
        pltpu.SemaphoreType.DMA,
        pltpu.SemaphoreType.DMA,
        pltpu.SemaphoreType.DMA,
        pltpu.SemaphoreType.DMA,
        pltpu.SemaphoreType.DMA,
        pltpu.SemaphoreType.DMA,
    ],
)
def _sc_gather(idx_hbm, table_hbm, out_hbm,
               idx_v, rows_v,
               g0, g1, g2, o0, o1, o2):
    wid = lax.axis_index("s") * NC + lax.axis_index("c")
    gsem = (g0, g1, g2)
    osem = (o0, o1, o2)

    pltpu.sync_copy(idx_hbm.at[wid], idx_v)

    def pos(k):
        return (k // CPP) * NW + wid

    def valid(k):
        return pos(k) < NPOS

    def gather_start(k, b):
        pltpu.make_async_copy(
            table_hbm.at[idx_v.at[k // CPP, k % CPP]], rows_v.at[b], gsem[b]
        ).start()

    def gather_wait(b):
        pltpu.make_async_copy(
            table_hbm.at[pl.ds(0, CHUNK)], rows_v.at[b], gsem[b]
        ).wait()

    def out_start(k, b):
        p = pos(k)
        b0 = (k % CPP) * CHUNK
        pltpu.make_async_copy(
            rows_v.at[b], out_hbm.at[p, pl.ds(b0, CHUNK)], osem[b]
        ).start()

    def out_wait(b):
        pltpu.make_async_copy(
            table_hbm.at[pl.ds(0, CHUNK)], rows_v.at[b], osem[b]
        ).wait()

    for b in range(NBUF):
        gather_start(b, b)

    def body(kk, carry):
        k0 = kk * NBUF
        for b in range(NBUF):

            @pl.when(valid(k0 + b))
            def _():
                gather_wait(b)
                out_start(k0 + b, b)

        for b in range(NBUF):
            k2 = k0 + b + NBUF

            @pl.when(jnp.logical_and(k2 < NCHUNK, valid(k2)))
            def _():
                out_wait(b)
                gather_start(k2, b)

        return carry

    lax.fori_loop(0, NCHUNK // NBUF, body, 0)
    for b in range(NBUF):
        out_wait(b)


def kernel(idx, table):
    idx_t = idx.astype(jnp.int32).T
    idx_t = jnp.pad(idx_t, ((0, NW * PPW - NPOS), (0, 0)))
    idx_r = idx_t.reshape(PPW, NW, NIMG).transpose(1, 0, 2)
    idx_r = idx_r.reshape(NW, PPW, CPP, CHUNK)
    out_t = _sc_gather(idx_r, table)
    return jnp.transpose(out_t, (1, 0, 2))
